# baseline (speedup 1.0000x reference)
import functools

import jax
import jax.numpy as jnp
from jax import lax
from jax.experimental import pallas as pl
from jax.experimental.pallas import tpu as pltpu
from jax.experimental.pallas import tpu_sc as plsc

NNODE = 10000
NGRAPH = 16
NDIM = 128
NTOT = NNODE + NGRAPH
RP = 10240
ZR = RP // 16

NWORK = 32
CHUNK = 128
NBUF = 2
BR = 1024


def _make_sc_agg(ch_per_w):
    mesh = plsc.VectorSubcoreMesh(core_axis_name="c", subcore_axis_name="s")

    @functools.partial(
        pl.kernel,
        out_type=jax.ShapeDtypeStruct((2, RP, NDIM), jnp.float32),
        mesh=mesh,
        compiler_params=pltpu.CompilerParams(use_tc_tiling_on_sc=False),
        scratch_types=(
            [pltpu.VMEM((2, CHUNK), jnp.int32) for _ in range(NBUF)] +
            [pltpu.VMEM((CHUNK, NDIM // 2), jnp.int32) for _ in range(NBUF)] +
            [pltpu.VMEM((CHUNK, NDIM), jnp.float32)] +
            [pltpu.VMEM_SHARED((RP, NDIM), jnp.float32)] +
            [pltpu.SemaphoreType.DMA for _ in range(2 * NBUF)]
        ),
    )
    def sc_agg(h_hbm, sd_hbm, zeros_hbm, out_hbm, *bufs):
        idx = bufs[:NBUF]
        rows = bufs[NBUF:2 * NBUF]
        rows_f = bufs[2 * NBUF]
        agg_sh = bufs[2 * NBUF + 1]
        gsem = bufs[2 * NBUF + 2:3 * NBUF + 2]
        isem = bufs[3 * NBUF + 2:]
        cid = lax.axis_index("c")
        sid = lax.axis_index("s")
        w = cid * 16 + sid
        pltpu.sync_copy(zeros_hbm, agg_sh.at[pl.ds(sid * ZR, ZR)])
        plsc.subcore_barrier()

        for b in range(NBUF):
            pltpu.async_copy(sd_hbm.at[w, b], idx[b], isem[b])
        for b in range(NBUF - 1):
            pltpu.make_async_copy(sd_hbm.at[w, b], idx[b], isem[b]).wait()
            pltpu.async_copy(h_hbm.at[idx[b].at[0]], rows[b], gsem[b])

        def body(p, carry):
            for b in range(NBUF):
                g = NBUF * p + b
                nb = (b + NBUF - 1) % NBUF
                pltpu.make_async_copy(h_hbm.at[idx[b].at[0]], rows[b],
                                      gsem[b]).wait()

                @pl.when(g + NBUF - 1 < ch_per_w)
                def _():
                    pltpu.make_async_copy(sd_hbm.at[w, g + NBUF - 1], idx[nb],
                                          isem[nb]).wait()
                    pltpu.async_copy(h_hbm.at[idx[nb].at[0]], rows[nb],
                                     gsem[nb])

                def conv(r4, carry2):
                    r = r4 * 4
                    for ro in range(4):
                        for c in range(NDIM // 32):
                            seg = rows[b][r + ro, pl.ds(16 * c, 16)]
                            lo = lax.bitcast_convert_type(seg << 16,
                                                          jnp.float32)
                            hi = lax.bitcast_convert_type(
                                seg & jnp.int32(-65536), jnp.float32)
                            rows_f[r + ro, pl.ds(32 * c, 16)] = lo
                            rows_f[r + ro, pl.ds(32 * c + 16, 16)] = hi
                    return carry2

                lax.fori_loop(0, CHUNK // 4, conv, 0)

                pltpu.sync_copy(rows_f, agg_sh.at[idx[b].at[1]], add=True)

                @pl.when(g + NBUF < ch_per_w)
                def _():
                    pltpu.async_copy(sd_hbm.at[w, g + NBUF], idx[b], isem[b])
            return carry

        lax.fori_loop(0, ch_per_w // NBUF, body, 0)
        plsc.subcore_barrier()
        pltpu.sync_copy(agg_sh.at[pl.ds(sid * ZR, ZR)],
                        out_hbm.at[cid, pl.ds(sid * ZR, ZR)])

    return sc_agg


def _mlp_body(h_ref, a_ref, m_ref, p_ref, vc_ref, w1_ref, b1_ref,
              w2_ref, b2_ref, o_ref, vo_ref, *rest, last):
    i = pl.program_id(0)
    z = h_ref[...] + a_ref[0] + a_ref[1]
    z = z + jnp.dot(m_ref[...], vc_ref[...], preferred_element_type=jnp.float32)
    t = jnp.dot(z, w1_ref[...], preferred_element_type=jnp.float32) + b1_ref[...]
    t = jnp.maximum(t, 0.0)
    y = jnp.dot(t, w2_ref[...], preferred_element_type=jnp.float32) + b2_ref[...]
    if not last:
        y = jnp.maximum(y, 0.0)
        rest[0][...] = y.astype(jnp.bfloat16)
    o_ref[...] = y
    part = lax.dot_general(p_ref[...], y, (((0,), (0,)), ((), ())),
                           preferred_element_type=jnp.float32)

    @pl.when(i == 0)
    def _():
        vo_ref[...] = jnp.zeros_like(vo_ref)

    vo_ref[...] += part


def _mlp(h, agg, m, p, vcat, w1t, b1, w2t, b2, last):
    grid = (RP // BR,)
    out_specs = [
        pl.BlockSpec((BR, NDIM), lambda i: (i, 0)),
        pl.BlockSpec((2 * NGRAPH, NDIM), lambda i: (0, 0)),
    ]
    out_shape = [
        jax.ShapeDtypeStruct((RP, NDIM), jnp.float32),
        jax.ShapeDtypeStruct((2 * NGRAPH, NDIM), jnp.float32),
    ]
    if not last:
        out_specs.append(pl.BlockSpec((BR, NDIM), lambda i: (i, 0)))
        out_shape.append(jax.ShapeDtypeStruct((RP, NDIM), jnp.bfloat16))
    return pl.pallas_call(
        functools.partial(_mlp_body, last=last),
        grid=grid,
        in_specs=[
            pl.BlockSpec((BR, NDIM), lambda i: (i, 0)),
            pl.BlockSpec((2, BR, NDIM), lambda i: (0, i, 0)),
            pl.BlockSpec((BR, 2 * NGRAPH), lambda i: (i, 0)),
            pl.BlockSpec((BR, 2 * NGRAPH), lambda i: (i, 0)),
            pl.BlockSpec((2 * NGRAPH, NDIM), lambda i: (0, 0)),
            pl.BlockSpec((NDIM, NDIM), lambda i: (0, 0)),
            pl.BlockSpec((1, NDIM), lambda i: (0, 0)),
            pl.BlockSpec((NDIM, NDIM), lambda i: (0, 0)),
            pl.BlockSpec((1, NDIM), lambda i: (0, 0)),
        ],
        out_specs=out_specs,
        out_shape=out_shape,
    )(h, agg, m, p, vcat, w1t, b1, w2t, b2)


def _vcat0_body(h_ref, p_ref, vo_ref):
    i = pl.program_id(0)

    @pl.when(i == 0)
    def _():
        vo_ref[...] = jnp.zeros_like(vo_ref)

    vo_ref[...] += lax.dot_general(p_ref[...], h_ref[...],
                                   (((0,), (0,)), ((), ())),
                                   preferred_element_type=jnp.float32)


def _vcat0(h, p):
    return pl.pallas_call(
        _vcat0_body,
        grid=(RP // BR,),
        in_specs=[
            pl.BlockSpec((BR, NDIM), lambda i: (i, 0)),
            pl.BlockSpec((BR, 2 * NGRAPH), lambda i: (i, 0)),
        ],
        out_specs=pl.BlockSpec((2 * NGRAPH, NDIM), lambda i: (0, 0)),
        out_shape=jax.ShapeDtypeStruct((2 * NGRAPH, NDIM), jnp.float32),
    )(h, p)


def _final_body(hv_ref, wl_ref, bl_ref, o_ref):
    z = jnp.maximum(hv_ref[...], 0.0)
    o_ref[...] = (jnp.dot(z, wl_ref[...], preferred_element_type=jnp.float32)
                  + bl_ref[...])


def _final(hv, wlt, bl):
    return pl.pallas_call(
        _final_body,
        out_shape=jax.ShapeDtypeStruct((NGRAPH, NDIM), jnp.float32),
    )(hv, wlt, bl)


def kernel(x, edge_index, batch, W1, b1, W2, b2, Wl, bl):
    n, d = x.shape
    e = edge_index.shape[1]
    idt = jnp.int32
    src = edge_index[0].astype(idt)
    dst = edge_index[1].astype(idt)
    ch_per_w = -(-e // (NWORK * CHUNK))
    ch_per_w = -(-ch_per_w // NBUF) * NBUF
    pad = NWORK * CHUNK * ch_per_w - e
    pk = jnp.arange(pad, dtype=idt)
    src = jnp.concatenate([src, (pk * 131) % jnp.int32(n)])
    dst = jnp.concatenate([dst, NTOT + pk % jnp.int32(RP - NTOT)])
    sd = jnp.stack([src.reshape(NWORK, ch_per_w, CHUNK),
                    dst.reshape(NWORK, ch_per_w, CHUNK)], axis=2)

    h = jnp.zeros((RP, NDIM), jnp.float32).at[:n].set(x)
    zeros = jnp.zeros((ZR, NDIM), jnp.float32)
    gids = jnp.arange(NGRAPH, dtype=idt)
    bpad = jnp.concatenate([batch.astype(idt), jnp.full((RP - n,), -1, idt)])
    m_real = (bpad[:, None] == gids[None, :]).astype(jnp.float32)
    rows_i = jnp.arange(RP, dtype=idt)
    m_virt = ((rows_i[:, None] - n) == gids[None, :]).astype(jnp.float32)
    m = jnp.concatenate([m_real, m_virt], axis=1)
    p = jnp.concatenate([m_virt, m_real], axis=1)

    w1t = jnp.swapaxes(W1, 1, 2)
    w2t = jnp.swapaxes(W2, 1, 2)
    nl = W1.shape[0]
    b1r = b1.reshape(nl, 1, NDIM)
    b2r = b2.reshape(nl, 1, NDIM)

    def swz(a):
        r = a.reshape(-1, NDIM // 32, 2, 16)
        sw = jnp.stack([r[:, :, 0, :], r[:, :, 1, :]], axis=-1)
        return lax.bitcast_convert_type(
            sw.reshape(-1, NDIM // 2, 2), jnp.int32)

    sc_agg = _make_sc_agg(ch_per_w)
    vcat = _vcat0(h, p)
    hbf = swz(h.astype(jnp.bfloat16))
    for l in range(nl):
        agg = sc_agg(hbf, sd, zeros)
        outs = _mlp(h, agg, m, p, vcat, w1t[l], b1r[l], w2t[l], b2r[l],
                    last=(l == nl - 1))
        h, vcat = outs[0], outs[1]
        if l < nl - 1:
            hbf = swz(outs[2])

    hv = lax.slice(vcat, (0, 0), (NGRAPH, NDIM))
    return _final(hv, Wl.T, bl.reshape(1, NDIM))


# speedup vs baseline: 2.0929x; 2.0929x over previous
import functools

import jax
import jax.numpy as jnp
from jax import lax
from jax.experimental import pallas as pl
from jax.experimental.pallas import tpu as pltpu
from jax.experimental.pallas import tpu_sc as plsc

NNODE = 10000
NGRAPH = 16
NDIM = 128
NTOT = NNODE + NGRAPH
RP = 10240
ZR = RP // 16

NWORK = 32
CHUNK = 128
NBUF = 2
BR = 1024


def _make_sc_agg(ch_per_w):
    mesh = plsc.VectorSubcoreMesh(core_axis_name="c", subcore_axis_name="s")

    @functools.partial(
        pl.kernel,
        out_type=jax.ShapeDtypeStruct((2, RP, NDIM), jnp.float32),
        mesh=mesh,
        scratch_types=(
            [pltpu.VMEM((2, CHUNK), jnp.int32) for _ in range(NBUF)] +
            [pltpu.VMEM((CHUNK, NDIM), jnp.float32) for _ in range(NBUF)] +
            [pltpu.VMEM_SHARED((RP, NDIM), jnp.float32)] +
            [pltpu.SemaphoreType.DMA for _ in range(2 * NBUF)]
        ),
    )
    def sc_agg(h_hbm, sd_hbm, zeros_hbm, out_hbm, *bufs):
        idx = bufs[:NBUF]
        rows = bufs[NBUF:2 * NBUF]
        agg_sh = bufs[2 * NBUF]
        gsem = bufs[2 * NBUF + 1:3 * NBUF + 1]
        isem = bufs[3 * NBUF + 1:]
        cid = lax.axis_index("c")
        sid = lax.axis_index("s")
        w = cid * 16 + sid
        pltpu.sync_copy(zeros_hbm, agg_sh.at[pl.ds(sid * ZR, ZR)])
        plsc.subcore_barrier()

        for b in range(NBUF):
            pltpu.async_copy(sd_hbm.at[w, b], idx[b], isem[b])
        for b in range(NBUF - 1):
            pltpu.make_async_copy(sd_hbm.at[w, b], idx[b], isem[b]).wait()
            pltpu.async_copy(h_hbm.at[idx[b].at[0]], rows[b], gsem[b])

        def body(p, carry):
            for b in range(NBUF):
                g = NBUF * p + b
                nb = (b + NBUF - 1) % NBUF
                pltpu.make_async_copy(h_hbm.at[idx[b].at[0]], rows[b],
                                      gsem[b]).wait()

                @pl.when(g + NBUF - 1 < ch_per_w)
                def _():
                    pltpu.make_async_copy(sd_hbm.at[w, g + NBUF - 1], idx[nb],
                                          isem[nb]).wait()
                    pltpu.async_copy(h_hbm.at[idx[nb].at[0]], rows[nb],
                                     gsem[nb])

                pltpu.sync_copy(rows[b], agg_sh.at[idx[b].at[1]], add=True)

                @pl.when(g + NBUF < ch_per_w)
                def _():
                    pltpu.async_copy(sd_hbm.at[w, g + NBUF], idx[b], isem[b])
            return carry

        lax.fori_loop(0, ch_per_w // NBUF, body, 0)
        plsc.subcore_barrier()
        pltpu.sync_copy(agg_sh.at[pl.ds(sid * ZR, ZR)],
                        out_hbm.at[cid, pl.ds(sid * ZR, ZR)])

    return sc_agg


def _mlp_body(h_ref, a_ref, m_ref, p_ref, vc_ref, w1_ref, b1_ref,
              w2_ref, b2_ref, o_ref, vo_ref, *, last):
    i = pl.program_id(0)
    z = h_ref[...] + a_ref[0] + a_ref[1]
    z = z + jnp.dot(m_ref[...], vc_ref[...], preferred_element_type=jnp.float32)
    t = jnp.dot(z, w1_ref[...], preferred_element_type=jnp.float32) + b1_ref[...]
    t = jnp.maximum(t, 0.0)
    y = jnp.dot(t, w2_ref[...], preferred_element_type=jnp.float32) + b2_ref[...]
    if not last:
        y = jnp.maximum(y, 0.0)
    o_ref[...] = y
    part = lax.dot_general(p_ref[...], y, (((0,), (0,)), ((), ())),
                           preferred_element_type=jnp.float32)

    @pl.when(i == 0)
    def _():
        vo_ref[...] = jnp.zeros_like(vo_ref)

    vo_ref[...] += part


def _mlp(h, agg, m, p, vcat, w1t, b1, w2t, b2, last):
    grid = (RP // BR,)
    return pl.pallas_call(
        functools.partial(_mlp_body, last=last),
        grid=grid,
        in_specs=[
            pl.BlockSpec((BR, NDIM), lambda i: (i, 0)),
            pl.BlockSpec((2, BR, NDIM), lambda i: (0, i, 0)),
            pl.BlockSpec((BR, 2 * NGRAPH), lambda i: (i, 0)),
            pl.BlockSpec((BR, 2 * NGRAPH), lambda i: (i, 0)),
            pl.BlockSpec((2 * NGRAPH, NDIM), lambda i: (0, 0)),
            pl.BlockSpec((NDIM, NDIM), lambda i: (0, 0)),
            pl.BlockSpec((1, NDIM), lambda i: (0, 0)),
            pl.BlockSpec((NDIM, NDIM), lambda i: (0, 0)),
            pl.BlockSpec((1, NDIM), lambda i: (0, 0)),
        ],
        out_specs=[
            pl.BlockSpec((BR, NDIM), lambda i: (i, 0)),
            pl.BlockSpec((2 * NGRAPH, NDIM), lambda i: (0, 0)),
        ],
        out_shape=[
            jax.ShapeDtypeStruct((RP, NDIM), jnp.float32),
            jax.ShapeDtypeStruct((2 * NGRAPH, NDIM), jnp.float32),
        ],
    )(h, agg, m, p, vcat, w1t, b1, w2t, b2)


def _vcat0_body(h_ref, p_ref, vo_ref):
    i = pl.program_id(0)

    @pl.when(i == 0)
    def _():
        vo_ref[...] = jnp.zeros_like(vo_ref)

    vo_ref[...] += lax.dot_general(p_ref[...], h_ref[...],
                                   (((0,), (0,)), ((), ())),
                                   preferred_element_type=jnp.float32)


def _vcat0(h, p):
    return pl.pallas_call(
        _vcat0_body,
        grid=(RP // BR,),
        in_specs=[
            pl.BlockSpec((BR, NDIM), lambda i: (i, 0)),
            pl.BlockSpec((BR, 2 * NGRAPH), lambda i: (i, 0)),
        ],
        out_specs=pl.BlockSpec((2 * NGRAPH, NDIM), lambda i: (0, 0)),
        out_shape=jax.ShapeDtypeStruct((2 * NGRAPH, NDIM), jnp.float32),
    )(h, p)


def _final_body(hv_ref, wl_ref, bl_ref, o_ref):
    z = jnp.maximum(hv_ref[...], 0.0)
    o_ref[...] = (jnp.dot(z, wl_ref[...], preferred_element_type=jnp.float32)
                  + bl_ref[...])


def _final(hv, wlt, bl):
    return pl.pallas_call(
        _final_body,
        out_shape=jax.ShapeDtypeStruct((NGRAPH, NDIM), jnp.float32),
    )(hv, wlt, bl)


def kernel(x, edge_index, batch, W1, b1, W2, b2, Wl, bl):
    n, d = x.shape
    e = edge_index.shape[1]
    idt = jnp.int32
    src = edge_index[0].astype(idt)
    dst = edge_index[1].astype(idt)
    ch_per_w = -(-e // (NWORK * CHUNK))
    ch_per_w = -(-ch_per_w // NBUF) * NBUF
    pad = NWORK * CHUNK * ch_per_w - e
    pk = jnp.arange(pad, dtype=idt)
    src = jnp.concatenate([src, (pk * 131) % jnp.int32(n)])
    dst = jnp.concatenate([dst, NTOT + pk % jnp.int32(RP - NTOT)])
    sd = jnp.stack([src.reshape(NWORK, ch_per_w, CHUNK),
                    dst.reshape(NWORK, ch_per_w, CHUNK)], axis=2)

    h = jnp.zeros((RP, NDIM), jnp.float32).at[:n].set(x)
    zeros = jnp.zeros((ZR, NDIM), jnp.float32)
    gids = jnp.arange(NGRAPH, dtype=idt)
    bpad = jnp.concatenate([batch.astype(idt), jnp.full((RP - n,), -1, idt)])
    m_real = (bpad[:, None] == gids[None, :]).astype(jnp.float32)
    rows_i = jnp.arange(RP, dtype=idt)
    m_virt = ((rows_i[:, None] - n) == gids[None, :]).astype(jnp.float32)
    m = jnp.concatenate([m_real, m_virt], axis=1)
    p = jnp.concatenate([m_virt, m_real], axis=1)

    w1t = jnp.swapaxes(W1, 1, 2)
    w2t = jnp.swapaxes(W2, 1, 2)
    nl = W1.shape[0]
    b1r = b1.reshape(nl, 1, NDIM)
    b2r = b2.reshape(nl, 1, NDIM)

    sc_agg = _make_sc_agg(ch_per_w)
    vcat = _vcat0(h, p)
    for l in range(nl):
        agg = sc_agg(h, sd, zeros)
        h, vcat = _mlp(h, agg, m, p, vcat, w1t[l], b1r[l], w2t[l], b2r[l],
                       last=(l == nl - 1))

    hv = lax.slice(vcat, (0, 0), (NGRAPH, NDIM))
    return _final(hv, Wl.T, bl.reshape(1, NDIM))
